# baseline (speedup 1.0000x reference)
import functools

import jax
import jax.numpy as jnp
from jax import lax
from jax.experimental import pallas as pl
from jax.experimental.pallas import tpu as pltpu
from jax.experimental.pallas import tpu_sc as plsc

B = 16
N = 20000
C = 85
K = 1000
CONF_BITS = 0x3E800000
NBKT = 4096
SBUF = 2048
SCH = 160
NSCH = N // SCH
ROWCH = 128
LANES = 16
NGRP = 64


def _body(x_hbm, out_hbm, scores_v, sbuf, sbuf2, hist, sba, sia, sbb, sib,
          tmp17, spans, rows_v, sem_g0, sem_g1):
    cid = lax.axis_index("c")
    sid = lax.axis_index("s")
    b = cid * 8 + sid // 2
    active = (sid % 2) == 0

    @pl.when(active)
    def _():
        ones = jnp.ones((LANES,), jnp.int32)
        zeros_i = jnp.zeros((LANES,), jnp.int32)
        zeros_f = jnp.zeros((LANES,), jnp.float32)
        iota = lax.iota(jnp.int32, LANES)
        base_row = b * N

        def clr_hist(k, _):
            hist[pl.ds(k * LANES, LANES)] = zeros_i
            return 0

        lax.fori_loop(0, NBKT // LANES, clr_hist, 0, unroll=False)

        def clr_surv(k, _):
            o = k * LANES
            sba[pl.ds(o, LANES)] = zeros_i
            sia[pl.ds(o, LANES)] = zeros_i
            sbb[pl.ds(o, LANES)] = zeros_i
            sib[pl.ds(o, LANES)] = zeros_i
            return 0

        lax.fori_loop(0, SBUF // LANES, clr_surv, 0, unroll=False)
        tmp17[pl.ds(0, LANES)] = jnp.full((LANES,), -1, jnp.int32)

        def p1_issue(ch, buf, sem):
            pltpu.async_copy(
                x_hbm.at[b, pl.ds(ch * SCH, SCH), :], buf, sem)

        def p1_consume(ch, buf, sem):
            pltpu.make_async_copy(
                x_hbm.at[b, pl.ds(0, SCH), :], buf, sem).wait()

            def p1_grp(j, _):
                ridx = j * LANES + iota
                cidx = jnp.full((LANES,), 4, jnp.int32)
                v = plsc.load_gather(
                    buf.at[pl.ds(0, SCH), pl.ds(0, C)], [ridx, cidx])
                scores_v[pl.ds(ch * SCH + j * LANES, LANES)] = v
                bits = plsc.bitcast(v, jnp.int32)
                d = jnp.right_shift(bits, 18)
                plsc.addupdate_scatter(hist.at[pl.ds(0, NBKT)], [d], ones)
                return 0

            lax.fori_loop(0, SCH // LANES, p1_grp, 0, unroll=False)

        p1_issue(0, sbuf, sem_g0)

        def p1_chunk(ch, _):
            par = ch - (ch // 2) * 2

            @pl.when((par == 0) & (ch + 1 < NSCH))
            def _():
                p1_issue(ch + 1, sbuf2, sem_g1)

            @pl.when((par == 1) & (ch + 1 < NSCH))
            def _():
                p1_issue(ch + 1, sbuf, sem_g0)

            @pl.when(par == 0)
            def _():
                p1_consume(ch, sbuf, sem_g0)

            @pl.when(par == 1)
            def _():
                p1_consume(ch, sbuf2, sem_g1)

            return 0

        lax.fori_loop(0, NSCH, p1_chunk, 0, unroll=False)

        def p2_chunk(k, carry):
            carry_cnt, v1vec = carry
            kk = NBKT // LANES - 1 - k
            h16 = hist[pl.ds(kk * LANES, LANES)]
            incl = plsc.cumsum(h16)
            tot = jnp.sum(h16)
            above = carry_cnt + tot - incl
            cond = (above < K) & (above + h16 >= K)
            i = plsc.all_reduce_ffs(cond)
            v1vec = jnp.where((v1vec < 0) & (i < LANES),
                              kk * LANES + i, v1vec)
            return carry_cnt + tot, v1vec

        _, v1vec = lax.fori_loop(
            0, NBKT // LANES, p2_chunk,
            (jnp.int32(0), jnp.full((LANES,), -1, jnp.int32)), unroll=False)
        t_floor = jnp.left_shift(v1vec, 18)

        def p3_chunk(j, ptr):
            v = scores_v[pl.ds(j * LANES, LANES)]
            bits = plsc.bitcast(v, jnp.int32)
            keep = bits >= t_floor
            k32 = jnp.where(keep, 1, 0)
            rank = plsc.cumsum(k32) - k32
            pos = ptr + rank
            m = keep & (pos < SBUF)
            plsc.store_scatter(sba.at[pl.ds(0, SBUF)], [pos], bits, mask=m)
            plsc.store_scatter(sia.at[pl.ds(0, SBUF)], [pos],
                               j * LANES + iota, mask=m)
            return jnp.minimum(ptr + jnp.sum(k32), SBUF)

        lax.fori_loop(0, N // LANES, p3_chunk, jnp.int32(0), unroll=False)

        def radix_pass(srcb, srci, dstb, dsti, shift):
            def clr(k, _):
                hist[pl.ds(k * LANES, LANES)] = zeros_i
                return 0

            lax.fori_loop(0, 1024 // LANES, clr, 0, unroll=False)

            def hcount(ch, _):
                bits = srcb[pl.ds(ch * LANES, LANES)]
                kp = ((1 << 30) - 1) - bits
                d = jnp.right_shift(kp, shift) & 1023
                plsc.addupdate_scatter(hist.at[pl.ds(0, 1024)], [d], ones)
                return 0

            lax.fori_loop(0, SBUF // LANES, hcount, 0, unroll=False)

            def scan(k, carry):
                h16 = hist[pl.ds(k * LANES, LANES)]
                incl = plsc.cumsum(h16)
                hist[pl.ds(k * LANES, LANES)] = incl - h16 + carry
                return carry + jnp.sum(h16)

            lax.fori_loop(0, 1024 // LANES, scan, jnp.int32(0), unroll=False)

            def place(ch, _):
                o = ch * LANES
                bits = srcb[pl.ds(o, LANES)]
                kp = ((1 << 30) - 1) - bits
                d = jnp.right_shift(kp, shift) & 1023
                key = d * LANES + iota
                sk, sv = plsc.sort_key_val(key, iota)
                sd = jnp.right_shift(sk, 4)
                tmp17[pl.ds(1, LANES)] = sd
                prev = tmp17[pl.ds(0, LANES)]
                start = sd != prev
                seg = plsc.cummax(jnp.where(start, iota, 0))
                w = iota - seg
                cur = plsc.load_gather(hist.at[pl.ds(0, 1024)], [sd])
                pos = cur + w
                bv = plsc.load_gather(srcb.at[pl.ds(o, LANES)], [sv])
                iv = plsc.load_gather(srci.at[pl.ds(o, LANES)], [sv])
                plsc.store_scatter(dstb.at[pl.ds(0, SBUF)], [pos], bv)
                plsc.store_scatter(dsti.at[pl.ds(0, SBUF)], [pos], iv)
                plsc.store_scatter(hist.at[pl.ds(0, 1024)], [sd], pos + 1)
                return 0

            lax.fori_loop(0, SBUF // LANES, place, 0, unroll=False)

        radix_pass(sba, sia, sbb, sib, 0)
        radix_pass(sbb, sib, sba, sia, 10)
        radix_pass(sba, sia, sbb, sib, 20)

        def rcount(k, acc):
            bits = sbb[pl.ds(k * LANES, LANES)]
            gpos = k * LANES + iota
            return acc + jnp.where((bits > CONF_BITS) & (gpos < K), 1, 0)

        racc = lax.fori_loop(0, (K + LANES - 1) // LANES, rcount,
                             zeros_i, unroll=False)
        r = jnp.sum(racc)

        def issue16(t, sem):
            def isb(u, _):
                k = t * LANES + u
                ridx = sib[pl.ds(k, LANES)][0]
                blk = pl.multiple_of((ridx // 8) * 8, 8)
                slot = k - (k // 32) * 32
                pltpu.async_copy(
                    x_hbm.at[b, pl.ds(blk, 8), :],
                    spans.at[pl.ds(slot * 8, 8), :], sem)
                return 0

            lax.fori_loop(0, LANES, isb, 0, unroll=False)

        def drain16(t, sem):
            def drb(u, _):
                k = t * LANES + u
                slot = k - (k // 32) * 32
                pltpu.make_async_copy(
                    x_hbm.at[b, pl.ds(0, 8), :],
                    spans.at[pl.ds(slot * 8, 8), :], sem).wait()
                ridx = sib[pl.ds(k, LANES)][0]
                q = ridx - (ridx // 8) * 8
                src = spans.at[slot * 8 + q]
                dst = rows_v.at[k - (k // ROWCH) * ROWCH]

                def cp(j, _):
                    dst[pl.ds(j * LANES, LANES)] = src[pl.ds(j * LANES,
                                                             LANES)]
                    return 0

                lax.fori_loop(0, 5, cp, 0, unroll=True)
                dst[pl.ds(C - LANES, LANES)] = src[pl.ds(C - LANES, LANES)]
                return 0

            lax.fori_loop(0, LANES, drb, 0, unroll=False)

        def zero_tail(g, rows_n):
            @pl.when(r < g * ROWCH + rows_n)
            def _():
                def zrow(m, _):
                    @pl.when(g * ROWCH + m >= r)
                    def _():
                        dst = rows_v.at[m]

                        def zp(j, _):
                            dst[pl.ds(j * LANES, LANES)] = zeros_f
                            return 0

                        lax.fori_loop(0, 5, zp, 0, unroll=True)
                        dst[pl.ds(C - LANES, LANES)] = zeros_f

                    return 0

                lax.fori_loop(0, rows_n, zrow, 0, unroll=False)

        def write_chunk_dyn(g):
            zero_tail(g, ROWCH)
            off = pl.multiple_of(g * ROWCH, 8)
            pltpu.sync_copy(rows_v.at[pl.ds(0, ROWCH), :],
                            out_hbm.at[b, pl.ds(off, ROWCH), :])

        issue16(0, sem_g0)

        def pipe(t, _):
            par = t - (t // 2) * 2

            @pl.when((t < NGRP) & (par == 1))
            def _():
                issue16(t, sem_g1)

            @pl.when((t < NGRP) & (par == 0))
            def _():
                issue16(t, sem_g0)

            @pl.when(par == 1)
            def _():
                drain16(t - 1, sem_g0)

            @pl.when(par == 0)
            def _():
                drain16(t - 1, sem_g1)

            tm8 = t - (t // 8) * 8

            @pl.when((tm8 == 0) & (t < NGRP))
            def _():
                write_chunk_dyn(t // 8 - 1)

            return 0

        lax.fori_loop(1, NGRP + 1, pipe, 0, unroll=False)

        last_rows = K - (K // ROWCH) * ROWCH + (ROWCH if K % ROWCH == 0
                                                else 0)
        zero_tail(jnp.int32(K // ROWCH), last_rows)
        pltpu.sync_copy(
            rows_v.at[pl.ds(0, last_rows), :],
            out_hbm.at[b, pl.ds((K // ROWCH) * ROWCH, last_rows), :])


def _make_kernel():
    return pl.kernel(
        _body,
        out_type=jax.ShapeDtypeStruct((B, K, C), jnp.float32),
        mesh=plsc.VectorSubcoreMesh(core_axis_name="c", subcore_axis_name="s"),
        compiler_params=pltpu.CompilerParams(needs_layout_passes=False),
        scratch_types=[
            pltpu.VMEM((N,), jnp.float32),
            pltpu.VMEM((SCH, C), jnp.float32),
            pltpu.VMEM((SCH, C), jnp.float32),
            pltpu.VMEM((NBKT,), jnp.int32),
            pltpu.VMEM((SBUF,), jnp.int32),
            pltpu.VMEM((SBUF,), jnp.int32),
            pltpu.VMEM((SBUF,), jnp.int32),
            pltpu.VMEM((SBUF,), jnp.int32),
            pltpu.VMEM((17,), jnp.int32),
            pltpu.VMEM((32 * 8, C), jnp.float32),
            pltpu.VMEM((ROWCH, C), jnp.float32),
            pltpu.SemaphoreType.DMA,
            pltpu.SemaphoreType.DMA,
        ],
    )


def kernel(x):
    return _make_kernel()(x)


# speedup vs baseline: 1.2970x; 1.2970x over previous
import jax
import jax.numpy as jnp
from jax import lax
from jax.experimental import pallas as pl
from jax.experimental.pallas import tpu as pltpu
from jax.experimental.pallas import tpu_sc as plsc

B = 16
N = 20000
C = 85
K = 1000
HN = N // 2
CONF_BITS = 0x3E800000
NBKT = 4096
SBUF = 2048
HBUF = 1024
SCH = 400
NSCH = HN // SCH
ROWCH = 128
LANES = 16
GRP = 8
NGRP = 512 // GRP


def _body(x_hbm, out_hbm, scores_v, sbuf, hist, histp, sba, sia, sbb, sib,
          stg_b, stg_i, meta_v, tmp17, spans, rows_v,
          shr_hist, shr_bits, shr_idx, shr_meta, sem_g0, sem_g1):
    cid = lax.axis_index("c")
    sid = lax.axis_index("s")
    b = cid * 8 + sid // 2
    h = sid - (sid // 2) * 2
    is_a = h == 0
    hbase = h * HN

    ones = jnp.ones((LANES,), jnp.int32)
    zeros_i = jnp.zeros((LANES,), jnp.int32)
    zeros_f = jnp.zeros((LANES,), jnp.float32)
    iota = lax.iota(jnp.int32, LANES)

    def clr_hist(k, _):
        hist[pl.ds(k * LANES, LANES)] = zeros_i
        return 0

    lax.fori_loop(0, NBKT // LANES, clr_hist, 0, unroll=False)

    def clr_surv(k, _):
        o = k * LANES
        sba[pl.ds(o, LANES)] = zeros_i
        sia[pl.ds(o, LANES)] = zeros_i
        sbb[pl.ds(o, LANES)] = zeros_i
        sib[pl.ds(o, LANES)] = zeros_i
        return 0

    lax.fori_loop(0, SBUF // LANES, clr_surv, 0, unroll=False)
    tmp17[pl.ds(0, LANES)] = jnp.full((LANES,), -1, jnp.int32)

    def p1_chunk(ch, _):
        pltpu.sync_copy(
            x_hbm.at[b, pl.ds(hbase + ch * SCH, SCH), :], sbuf)

        def p1_grp(j, _):
            ridx = j * LANES + iota
            cidx = jnp.full((LANES,), 4, jnp.int32)
            v = plsc.load_gather(
                sbuf.at[pl.ds(0, SCH), pl.ds(0, C)], [ridx, cidx])
            scores_v[pl.ds(ch * SCH + j * LANES, LANES)] = v
            bits = plsc.bitcast(v, jnp.int32)
            d = jnp.right_shift(bits, 18)
            plsc.addupdate_scatter(hist.at[pl.ds(0, NBKT)], [d], ones)
            return 0

        lax.fori_loop(0, SCH // LANES, p1_grp, 0, unroll=False)
        return 0

    lax.fori_loop(0, NSCH, p1_chunk, 0, unroll=False)

    pltpu.sync_copy(hist, shr_hist.at[sid])
    plsc.subcore_barrier()
    pltpu.sync_copy(shr_hist.at[sid + 1 - 2 * h], histp)

    def addh(k, _):
        o = k * LANES
        hist[pl.ds(o, LANES)] = hist[pl.ds(o, LANES)] + histp[pl.ds(o, LANES)]
        return 0

    lax.fori_loop(0, NBKT // LANES, addh, 0, unroll=False)

    def p2_chunk(k, carry):
        carry_cnt, v1vec = carry
        kk = NBKT // LANES - 1 - k
        h16 = hist[pl.ds(kk * LANES, LANES)]
        incl = plsc.cumsum(h16)
        tot = jnp.sum(h16)
        above = carry_cnt + tot - incl
        cond = (above < K) & (above + h16 >= K)
        i = plsc.all_reduce_ffs(cond)
        v1vec = jnp.where((v1vec < 0) & (i < LANES), kk * LANES + i, v1vec)
        return carry_cnt + tot, v1vec

    _, v1vec = lax.fori_loop(
        0, NBKT // LANES, p2_chunk,
        (jnp.int32(0), jnp.full((LANES,), -1, jnp.int32)), unroll=False)
    t_floor = jnp.left_shift(v1vec, 18)

    def p3_chunk(j, ptr):
        v = scores_v[pl.ds(j * LANES, LANES)]
        bits = plsc.bitcast(v, jnp.int32)
        keep = bits >= t_floor
        k32 = jnp.where(keep, 1, 0)
        rank = plsc.cumsum(k32) - k32
        pos = ptr + rank
        m = keep & (pos < HBUF)
        plsc.store_scatter(sba.at[pl.ds(0, SBUF)], [pos], bits, mask=m)
        plsc.store_scatter(sia.at[pl.ds(0, SBUF)], [pos],
                           hbase + j * LANES + iota, mask=m)
        return jnp.minimum(ptr + jnp.sum(k32), HBUF)

    s_own = lax.fori_loop(0, HN // LANES, p3_chunk, jnp.int32(0),
                          unroll=False)

    meta_v[pl.ds(0, LANES)] = jnp.full((LANES,), s_own, jnp.int32)
    pltpu.sync_copy(sba.at[pl.ds(0, HBUF)], shr_bits.at[sid])
    pltpu.sync_copy(sia.at[pl.ds(0, HBUF)], shr_idx.at[sid])
    pltpu.sync_copy(meta_v, shr_meta.at[sid])
    plsc.subcore_barrier()

    @pl.when(is_a)
    def _():
        pltpu.sync_copy(shr_bits.at[sid + 1], stg_b)
        pltpu.sync_copy(shr_idx.at[sid + 1], stg_i)
        pltpu.sync_copy(shr_meta.at[sid + 1], meta_v)
        s_b = meta_v[pl.ds(0, LANES)][0]

        def app(k, _):
            lane = k * LANES + iota
            pos = s_own + lane
            m = (lane < s_b) & (pos < SBUF)
            plsc.store_scatter(sba.at[pl.ds(0, SBUF)], [pos],
                               stg_b[pl.ds(k * LANES, LANES)], mask=m)
            plsc.store_scatter(sia.at[pl.ds(0, SBUF)], [pos],
                               stg_i[pl.ds(k * LANES, LANES)], mask=m)
            return 0

        lax.fori_loop(0, HBUF // LANES, app, 0, unroll=False)

        def radix_pass(srcb, srci, dstb, dsti, shift):
            def clr(k, _):
                hist[pl.ds(k * LANES, LANES)] = zeros_i
                return 0

            lax.fori_loop(0, 1024 // LANES, clr, 0, unroll=False)

            def hcount(ch, _):
                bits = srcb[pl.ds(ch * LANES, LANES)]
                kp = ((1 << 30) - 1) - bits
                d = jnp.right_shift(kp, shift) & 1023
                plsc.addupdate_scatter(hist.at[pl.ds(0, 1024)], [d], ones)
                return 0

            lax.fori_loop(0, SBUF // LANES, hcount, 0, unroll=False)

            def scan(k, carry):
                h16 = hist[pl.ds(k * LANES, LANES)]
                incl = plsc.cumsum(h16)
                hist[pl.ds(k * LANES, LANES)] = incl - h16 + carry
                return carry + jnp.sum(h16)

            lax.fori_loop(0, 1024 // LANES, scan, jnp.int32(0), unroll=False)

            def place(ch, _):
                o = ch * LANES
                bits = srcb[pl.ds(o, LANES)]
                kp = ((1 << 30) - 1) - bits
                d = jnp.right_shift(kp, shift) & 1023
                key = d * LANES + iota
                sk, sv = plsc.sort_key_val(key, iota)
                sd = jnp.right_shift(sk, 4)
                tmp17[pl.ds(1, LANES)] = sd
                prev = tmp17[pl.ds(0, LANES)]
                start = sd != prev
                seg = plsc.cummax(jnp.where(start, iota, 0))
                w = iota - seg
                cur = plsc.load_gather(hist.at[pl.ds(0, 1024)], [sd])
                pos = cur + w
                bv = plsc.load_gather(srcb.at[pl.ds(o, LANES)], [sv])
                iv = plsc.load_gather(srci.at[pl.ds(o, LANES)], [sv])
                plsc.store_scatter(dstb.at[pl.ds(0, SBUF)], [pos], bv)
                plsc.store_scatter(dsti.at[pl.ds(0, SBUF)], [pos], iv)
                plsc.store_scatter(hist.at[pl.ds(0, 1024)], [sd], pos + 1)
                return 0

            lax.fori_loop(0, SBUF // LANES, place, 0, unroll=False)

        radix_pass(sba, sia, sbb, sib, 0)
        radix_pass(sbb, sib, sba, sia, 10)
        radix_pass(sba, sia, sbb, sib, 20)

        def rcount(k, acc):
            bits = sbb[pl.ds(k * LANES, LANES)]
            gpos = k * LANES + iota
            return acc + jnp.where((bits > CONF_BITS) & (gpos < K), 1, 0)

        racc = lax.fori_loop(0, (K + LANES - 1) // LANES, rcount,
                             zeros_i, unroll=False)
        meta_v[pl.ds(0, LANES)] = racc * 0 + jnp.sum(racc)

        pltpu.sync_copy(sib.at[pl.ds(0, HBUF)], shr_idx.at[sid])
        pltpu.sync_copy(meta_v, shr_meta.at[sid])

    plsc.subcore_barrier()

    @pl.when(jnp.logical_not(is_a))
    def _():
        pltpu.sync_copy(shr_idx.at[sid - 1], sib.at[pl.ds(0, HBUF)])
        pltpu.sync_copy(shr_meta.at[sid - 1], meta_v)

    r = meta_v[pl.ds(0, LANES)][0]

    rank0 = h * 512

    def issueg(t, sem):
        def isb(u, _):
            k = rank0 + t * GRP + u
            ridx = sib[pl.ds(k, LANES)][0]
            blk = pl.multiple_of((ridx // 8) * 8, 8)
            slot = k - (k // 16) * 16
            pltpu.async_copy(
                x_hbm.at[b, pl.ds(blk, 8), :],
                spans.at[pl.ds(slot * 8, 8), :], sem)
            return 0

        lax.fori_loop(0, GRP, isb, 0, unroll=False)

    def draing(t, sem):
        def drb(u, _):
            k = rank0 + t * GRP + u
            slot = k - (k // 16) * 16
            pltpu.make_async_copy(
                x_hbm.at[b, pl.ds(0, 8), :],
                spans.at[pl.ds(slot * 8, 8), :], sem).wait()
            ridx = sib[pl.ds(k, LANES)][0]
            q = ridx - (ridx // 8) * 8
            src = spans.at[slot * 8 + q]
            dst = rows_v.at[k - (k // ROWCH) * ROWCH]

            def cp(j, _):
                dst[pl.ds(j * LANES, LANES)] = src[pl.ds(j * LANES, LANES)]
                return 0

            lax.fori_loop(0, 5, cp, 0, unroll=True)
            dst[pl.ds(C - LANES, LANES)] = src[pl.ds(C - LANES, LANES)]
            return 0

        lax.fori_loop(0, GRP, drb, 0, unroll=False)

    def zero_tail(g, rows_n):
        @pl.when(r < g * ROWCH + rows_n)
        def _():
            def zrow(m, _):
                @pl.when(g * ROWCH + m >= r)
                def _():
                    dst = rows_v.at[m]

                    def zp(j, _):
                        dst[pl.ds(j * LANES, LANES)] = zeros_f
                        return 0

                    lax.fori_loop(0, 5, zp, 0, unroll=True)
                    dst[pl.ds(C - LANES, LANES)] = zeros_f

                return 0

            lax.fori_loop(0, rows_n, zrow, 0, unroll=False)

    def write_chunk_dyn(g):
        zero_tail(g, ROWCH)
        off = pl.multiple_of(g * ROWCH, 8)
        pltpu.sync_copy(rows_v.at[pl.ds(0, ROWCH), :],
                        out_hbm.at[b, pl.ds(off, ROWCH), :])

    gr_per_chunk = ROWCH // GRP

    issueg(0, sem_g0)

    def pipe(t, _):
        par = t - (t // 2) * 2

        @pl.when((t < NGRP) & (par == 1))
        def _():
            issueg(t, sem_g1)

        @pl.when((t < NGRP) & (par == 0))
        def _():
            issueg(t, sem_g0)

        @pl.when(par == 1)
        def _():
            draing(t - 1, sem_g0)

        @pl.when(par == 0)
        def _():
            draing(t - 1, sem_g1)

        tmc = t - (t // gr_per_chunk) * gr_per_chunk

        @pl.when((tmc == 0) & (t < NGRP))
        def _():
            write_chunk_dyn(h * 4 + t // gr_per_chunk - 1)

        return 0

    lax.fori_loop(1, NGRP + 1, pipe, 0, unroll=False)

    @pl.when(is_a)
    def _():
        zero_tail(jnp.int32(3), ROWCH)
        pltpu.sync_copy(rows_v.at[pl.ds(0, ROWCH), :],
                        out_hbm.at[b, pl.ds(3 * ROWCH, ROWCH), :])

    last_rows = K - (K // ROWCH) * ROWCH

    @pl.when(jnp.logical_not(is_a))
    def _():
        zero_tail(jnp.int32(K // ROWCH), last_rows)
        pltpu.sync_copy(
            rows_v.at[pl.ds(0, last_rows), :],
            out_hbm.at[b, pl.ds((K // ROWCH) * ROWCH, last_rows), :])


def _make_kernel():
    return pl.kernel(
        _body,
        out_type=jax.ShapeDtypeStruct((B, K, C), jnp.float32),
        mesh=plsc.VectorSubcoreMesh(core_axis_name="c", subcore_axis_name="s"),
        compiler_params=pltpu.CompilerParams(needs_layout_passes=False),
        scratch_types=[
            pltpu.VMEM((HN,), jnp.float32),
            pltpu.VMEM((SCH, C), jnp.float32),
            pltpu.VMEM((NBKT,), jnp.int32),
            pltpu.VMEM((NBKT,), jnp.int32),
            pltpu.VMEM((SBUF,), jnp.int32),
            pltpu.VMEM((SBUF,), jnp.int32),
            pltpu.VMEM((SBUF,), jnp.int32),
            pltpu.VMEM((SBUF,), jnp.int32),
            pltpu.VMEM((HBUF,), jnp.int32),
            pltpu.VMEM((HBUF,), jnp.int32),
            pltpu.VMEM((LANES,), jnp.int32),
            pltpu.VMEM((17,), jnp.int32),
            pltpu.VMEM((16 * 8, C), jnp.float32),
            pltpu.VMEM((ROWCH, C), jnp.float32),
            pltpu.VMEM_SHARED((16, NBKT), jnp.int32),
            pltpu.VMEM_SHARED((16, HBUF), jnp.int32),
            pltpu.VMEM_SHARED((16, HBUF), jnp.int32),
            pltpu.VMEM_SHARED((16, LANES), jnp.int32),
            pltpu.SemaphoreType.DMA,
            pltpu.SemaphoreType.DMA,
        ],
    )


def kernel(x):
    return _make_kernel()(x)
